# baseline (speedup 1.0000x reference)
import functools

import jax
import jax.numpy as jnp
from jax import lax
from jax.experimental import pallas as pl
from jax.experimental.pallas import tpu as pltpu
from jax.experimental.pallas import tpu_sc as plsc

C = 64
NPTS = 2048
SX, SY, SZ = 100, 88, 80
VOL = SX * SY * SZ
NROWS = C * SY * SZ
PADVOL = SY * SZ * 128
NC, NS = 2, 16
NW = NC * NS
PPW = NPTS // NW
NV = PPW // 16
NROW = 32
LOSS_SCALE = 100.0 / (4.0 * NPTS)

ROWS_PER_TILE = NROWS // NW
CHROWS = 176
NCHUNK = ROWS_PER_TILE // CHROWS
CHWORDS = CHROWS * SX
ROW_OFFS = (0, 16, 32, 48, 64, 80, 84)


def _relayout_body(fixt_hbm, movt_hbm, fixo_hbm, movo_hbm,
                   in0, in1, out0, out1, sem_in, sem_out):
    cid = lax.axis_index("c")
    sid = lax.axis_index("s")
    wid = cid * NS + sid
    row0 = wid * ROWS_PER_TILE

    def repack(src_buf, dst_buf):
        UR = 4

        @plsc.parallel_loop(0, CHROWS, step=UR, unroll=4)
        def _rows(r0):
            vals = [
                src_buf[r0 + u, pl.ds(off, 16)]
                for u in range(UR)
                for off in ROW_OFFS
            ]
            k = 0
            for u in range(UR):
                o = (r0 + u) * SX
                for off in ROW_OFFS:
                    dst_buf[pl.ds(o + off, 16)] = vals[k]
                    k += 1

    for src, dst in ((fixt_hbm, fixo_hbm), (movt_hbm, movo_hbm)):
        def start_in(c, b):
            pltpu.async_copy(src.at[pl.ds(row0 + c * CHROWS, CHROWS)], b, sem_in)

        def wait_in(b):
            pltpu.make_async_copy(src.at[pl.ds(row0, CHROWS)], b, sem_in).wait()

        def start_out(c, b):
            pltpu.async_copy(
                b, dst.at[pl.ds((row0 + c * CHROWS) * SX, CHWORDS)], sem_out
            )

        def wait_out(b):
            pltpu.make_async_copy(
                b, dst.at[pl.ds(row0, CHWORDS)], sem_out
            ).wait()

        start_in(0, in0)

        def pair(m, _):
            c0 = 2 * m
            wait_in(in0)
            start_in(c0 + 1, in1)

            @pl.when(m > 0)
            def _():
                wait_out(out0)

            start_out(c0, out0)
            wait_in(in1)

            @pl.when(c0 + 2 < NCHUNK)
            def _():
                start_in(c0 + 2, in0)

            @pl.when(m > 0)
            def _():
                wait_out(out1)

            start_out(c0 + 1, out1)
            return 0

        lax.fori_loop(0, NCHUNK // 2, pair, 0)
        wait_out(out0)
        wait_out(out1)


def _sqrt16(x):
    i = plsc.bitcast(x, jnp.int32)
    y = plsc.bitcast(jnp.int32(0x5F3759DF) - (i >> 1), jnp.float32)
    for _ in range(3):
        y = y * (1.5 - 0.5 * x * y * y)
    return x * y


def _gather_body(fix_hbm, mov_hbm, pts_hbm,
                 parts_hbm, pos_hbm, neg_hbm,
                 pts_v, idx_v, rows_v, dis_v, part_v, red_v, shared, sem):
    cid = lax.axis_index("c")
    sid = lax.axis_index("s")
    wid = cid * NS + sid
    base = wid * PPW

    for r in range(9):
        pltpu.sync_copy(pts_hbm.at[pl.ds(r * NPTS + base, PPW)], pts_v.at[r])

    svecs = []
    for g in range(3):
        for i in range(NV):
            sl = pl.ds(i * 16, 16)
            x = lax.rem(pts_v[3 * g + 0, sl], SX)
            y = lax.rem(pts_v[3 * g + 1, sl], SY)
            z = lax.rem(pts_v[3 * g + 2, sl], SZ)
            svecs.append((y * SZ + z) * SX + x)

    def build_row(j, sv):
        c0 = (2 * j) * VOL
        for g in range(3):
            row = idx_v.at[g, j]
            for half in range(2):
                coff = c0 + half * VOL
                for i in range(NV):
                    row[pl.ds(half * PPW + i * 16, 16)] = sv[NV * g + i] + coff
        return sv

    lax.fori_loop(0, NROW, build_row, tuple(svecs))

    def fire(j, _):
        pltpu.async_copy(fix_hbm.at[idx_v.at[0, j]], rows_v.at[0, j], sem)
        pltpu.async_copy(mov_hbm.at[idx_v.at[1, j]], rows_v.at[1, j], sem)
        pltpu.async_copy(mov_hbm.at[idx_v.at[2, j]], rows_v.at[2, j], sem)
        return 0

    lax.fori_loop(0, NROW, fire, 0)

    def drain(j, _):
        pltpu.make_async_copy(fix_hbm.at[idx_v.at[0, j]], rows_v.at[0, j], sem).wait()
        pltpu.make_async_copy(mov_hbm.at[idx_v.at[1, j]], rows_v.at[1, j], sem).wait()
        pltpu.make_async_copy(mov_hbm.at[idx_v.at[2, j]], rows_v.at[2, j], sem).wait()
        return 0

    lax.fori_loop(0, NROW, drain, 0)

    def accum(j, accs):
        accs = list(accs)
        for half in range(2):
            for i in range(NV):
                sl = pl.ds(half * PPW + i * 16, 16)
                a = rows_v[0, j, sl]
                p = rows_v[1, j, sl]
                n = rows_v[2, j, sl]
                dp = a - p
                dn = a - n
                accs[i] = accs[i] + dp * dp
                accs[NV + i] = accs[NV + i] + dn * dn
        return tuple(accs)

    zeros = jnp.zeros((16,), jnp.float32)
    accs = lax.fori_loop(0, NROW, accum, (zeros,) * (2 * NV))

    psum = zeros
    nsum = zeros
    for i in range(NV):
        pos_d2 = accs[i]
        neg_d2 = accs[NV + i]
        dis_v[pl.ds(i * 16, 16)] = _sqrt16(pos_d2)
        neg_dis = _sqrt16(neg_d2)
        dis_v[pl.ds(PPW + i * 16, 16)] = neg_dis
        hinge = jnp.maximum(1.0 - neg_dis, 0.0)
        psum = psum + pos_d2
        nsum = nsum + hinge * hinge

    pltpu.sync_copy(dis_v.at[pl.ds(0, PPW)], pos_hbm.at[pl.ds(base, PPW)])
    pltpu.sync_copy(dis_v.at[pl.ds(PPW, PPW)], neg_hbm.at[pl.ds(base, PPW)])

    lane = jnp.arange(16, dtype=jnp.int32)
    part_v[...] = jnp.where(lane == 0, jnp.sum((psum + nsum) * LOSS_SCALE), 0.0)
    pltpu.sync_copy(part_v.at[pl.ds(0, 8)], shared.at[pl.ds(sid * 8, 8)])
    plsc.subcore_barrier()

    @pl.when(sid == 0)
    def _():
        pltpu.sync_copy(shared, red_v)
        mask = lax.rem(lane, 8) == 0
        tot = jnp.zeros((16,), jnp.float32)
        for k in range(8):
            v = red_v[pl.ds(k * 16, 16)]
            tot = tot + jnp.where(mask, v, 0.0)
        part_v[...] = jnp.where(lane == 0, jnp.sum(tot), 0.0)
        pltpu.sync_copy(part_v.at[pl.ds(0, 8)], parts_hbm.at[pl.ds(cid * 8, 8)])


@jax.jit
def _sc_call(fixt, movt, pts):
    mesh = plsc.VectorSubcoreMesh(core_axis_name="c", subcore_axis_name="s")
    relayout = functools.partial(
        pl.kernel,
        mesh=mesh,
        compiler_params=pltpu.CompilerParams(
            use_tc_tiling_on_sc=True, needs_layout_passes=False
        ),
        out_type=[
            jax.ShapeDtypeStruct((C * VOL,), jnp.float32),
            jax.ShapeDtypeStruct((C * VOL,), jnp.float32),
        ],
        scratch_types=[
            pltpu.VMEM((CHROWS, SX), jnp.float32),
            pltpu.VMEM((CHROWS, SX), jnp.float32),
            pltpu.VMEM((CHWORDS,), jnp.float32),
            pltpu.VMEM((CHWORDS,), jnp.float32),
            pltpu.SemaphoreType.DMA,
            pltpu.SemaphoreType.DMA,
        ],
    )(_relayout_body)
    fix_flat, mov_flat = relayout(fixt, movt)

    gather = functools.partial(
        pl.kernel,
        mesh=mesh,
        compiler_params=pltpu.CompilerParams(needs_layout_passes=False),
        out_type=[
            jax.ShapeDtypeStruct((NC * 8,), jnp.float32),
            jax.ShapeDtypeStruct((NPTS,), jnp.float32),
            jax.ShapeDtypeStruct((NPTS,), jnp.float32),
        ],
        scratch_types=[
            pltpu.VMEM((9, PPW), jnp.int32),
            pltpu.VMEM((3, NROW, 128), jnp.int32),
            pltpu.VMEM((3, NROW, 128), jnp.float32),
            pltpu.VMEM((2 * PPW,), jnp.float32),
            pltpu.VMEM((16,), jnp.float32),
            pltpu.VMEM((128,), jnp.float32),
            pltpu.VMEM_SHARED((128,), jnp.float32),
            pltpu.SemaphoreType.DMA,
        ],
    )(_gather_body)
    return gather(fix_flat, mov_flat, pts)


def kernel(fix_image_feature, moving_image_feature, fixed_points,
           positive_points, negative_points, x_shard, y_shard, z_shard):
    fixt = jnp.transpose(fix_image_feature, (0, 1, 3, 4, 2)).reshape(NROWS, SX)
    movt = jnp.transpose(moving_image_feature, (0, 1, 3, 4, 2)).reshape(NROWS, SX)
    pts = jnp.concatenate(
        [fixed_points.T, positive_points.T, negative_points.T], axis=0
    ).astype(jnp.int32).reshape(-1)
    parts, pos_dis, neg_dis = _sc_call(fixt, movt, pts)
    loss = parts[0] + parts[8]
    return loss, pos_dis, neg_dis


# speedup vs baseline: 1.1976x; 1.1976x over previous
import functools

import jax
import jax.numpy as jnp
from jax import lax
from jax.experimental import pallas as pl
from jax.experimental.pallas import tpu as pltpu
from jax.experimental.pallas import tpu_sc as plsc

C = 64
NPTS = 2048
SX, SY, SZ = 100, 88, 80
VOL = SX * SY * SZ
NROWS = C * SY * SZ
PADVOL = SY * SZ * 128
NC, NS = 2, 16
NW = NC * NS
PPW = NPTS // NW
NV = PPW // 16
NROW = 32
LOSS_SCALE = 100.0 / (4.0 * NPTS)

ROWS_PER_TILE = NROWS // NW
CHROWS = 320
NCHUNK = ROWS_PER_TILE // CHROWS
HROWS = CHROWS // 2
HWORDS = HROWS * SX
ROW_OFFS = (0, 16, 32, 48, 64, 80, 84)


def _relayout_body(fixt_hbm, movt_hbm, fixo_hbm, movo_hbm,
                   in0, in1, out0, out1, sem_in, sem_out0, sem_out1):
    cid = lax.axis_index("c")
    sid = lax.axis_index("s")
    wid = cid * NS + sid
    row0 = wid * ROWS_PER_TILE

    def repack(src_buf, half, dst_buf):
        UR = 4
        r_base = half * HROWS

        @plsc.parallel_loop(0, HROWS, step=UR, unroll=2)
        def _rows(r0):
            vals = [
                src_buf[r_base + r0 + u, pl.ds(off, 16)]
                for u in range(UR)
                for off in ROW_OFFS
            ]
            k = 0
            for u in range(UR):
                o = (r0 + u) * SX
                for off in ROW_OFFS:
                    dst_buf[pl.ds(o + off, 16)] = vals[k]
                    k += 1

    for src, dst in ((fixt_hbm, fixo_hbm), (movt_hbm, movo_hbm)):
        def start_in(c, b):
            pltpu.async_copy(src.at[pl.ds(row0 + c * CHROWS, CHROWS)], b, sem_in)

        def wait_in(b):
            pltpu.make_async_copy(src.at[pl.ds(row0, CHROWS)], b, sem_in).wait()

        def start_out(c, half, b, sem):
            off = (row0 + c * CHROWS + half * HROWS) * SX
            pltpu.async_copy(b, dst.at[pl.ds(off, HWORDS)], sem)

        def wait_out(b, sem):
            pltpu.make_async_copy(b, dst.at[pl.ds(row0, HWORDS)], sem).wait()

        def do_half(c, half, src_buf, out_buf, sem, first):
            if first:
                @pl.when(c > 0)
                def _():
                    wait_out(out_buf, sem)
            else:
                wait_out(out_buf, sem)
            repack(src_buf, half, out_buf)
            start_out(c, half, out_buf, sem)

        start_in(0, in0)

        def pair(m, _):
            c0 = 2 * m
            wait_in(in0)
            start_in(c0 + 1, in1)
            do_half(c0, 0, in0, out0, sem_out0, True)
            do_half(c0, 1, in0, out1, sem_out1, True)
            wait_in(in1)

            @pl.when(c0 + 2 < NCHUNK)
            def _():
                start_in(c0 + 2, in0)

            do_half(c0 + 1, 0, in1, out0, sem_out0, False)
            do_half(c0 + 1, 1, in1, out1, sem_out1, False)
            return 0

        lax.fori_loop(0, NCHUNK // 2, pair, 0)
        wait_out(out0, sem_out0)
        wait_out(out1, sem_out1)


def _sqrt16(x):
    i = plsc.bitcast(x, jnp.int32)
    y = plsc.bitcast(jnp.int32(0x5F3759DF) - (i >> 1), jnp.float32)
    for _ in range(3):
        y = y * (1.5 - 0.5 * x * y * y)
    return x * y


def _gather_body(fix_hbm, mov_hbm, pts_hbm,
                 parts_hbm, pos_hbm, neg_hbm,
                 pts_v, idx_v, rows_v, dis_v, part_v, red_v, shared, sem):
    cid = lax.axis_index("c")
    sid = lax.axis_index("s")
    wid = cid * NS + sid
    base = wid * PPW

    for r in range(9):
        pltpu.sync_copy(pts_hbm.at[pl.ds(r * NPTS + base, PPW)], pts_v.at[r])

    svecs = []
    for g in range(3):
        for i in range(NV):
            sl = pl.ds(i * 16, 16)
            x = lax.rem(pts_v[3 * g + 0, sl], SX)
            y = lax.rem(pts_v[3 * g + 1, sl], SY)
            z = lax.rem(pts_v[3 * g + 2, sl], SZ)
            svecs.append((y * SZ + z) * SX + x)

    def build_row(j, sv):
        c0 = (2 * j) * VOL
        for g in range(3):
            row = idx_v.at[g, j]
            for half in range(2):
                coff = c0 + half * VOL
                for i in range(NV):
                    row[pl.ds(half * PPW + i * 16, 16)] = sv[NV * g + i] + coff
        return sv

    lax.fori_loop(0, NROW, build_row, tuple(svecs))

    def fire(j, _):
        pltpu.async_copy(fix_hbm.at[idx_v.at[0, j]], rows_v.at[0, j], sem)
        pltpu.async_copy(mov_hbm.at[idx_v.at[1, j]], rows_v.at[1, j], sem)
        pltpu.async_copy(mov_hbm.at[idx_v.at[2, j]], rows_v.at[2, j], sem)
        return 0

    lax.fori_loop(0, NROW, fire, 0)

    def drain(j, _):
        pltpu.make_async_copy(fix_hbm.at[idx_v.at[0, j]], rows_v.at[0, j], sem).wait()
        pltpu.make_async_copy(mov_hbm.at[idx_v.at[1, j]], rows_v.at[1, j], sem).wait()
        pltpu.make_async_copy(mov_hbm.at[idx_v.at[2, j]], rows_v.at[2, j], sem).wait()
        return 0

    lax.fori_loop(0, NROW, drain, 0)

    def accum(j, accs):
        accs = list(accs)
        for half in range(2):
            for i in range(NV):
                sl = pl.ds(half * PPW + i * 16, 16)
                a = rows_v[0, j, sl]
                p = rows_v[1, j, sl]
                n = rows_v[2, j, sl]
                dp = a - p
                dn = a - n
                accs[i] = accs[i] + dp * dp
                accs[NV + i] = accs[NV + i] + dn * dn
        return tuple(accs)

    zeros = jnp.zeros((16,), jnp.float32)
    accs = lax.fori_loop(0, NROW, accum, (zeros,) * (2 * NV))

    psum = zeros
    nsum = zeros
    for i in range(NV):
        pos_d2 = accs[i]
        neg_d2 = accs[NV + i]
        dis_v[pl.ds(i * 16, 16)] = _sqrt16(pos_d2)
        neg_dis = _sqrt16(neg_d2)
        dis_v[pl.ds(PPW + i * 16, 16)] = neg_dis
        hinge = jnp.maximum(1.0 - neg_dis, 0.0)
        psum = psum + pos_d2
        nsum = nsum + hinge * hinge

    pltpu.sync_copy(dis_v.at[pl.ds(0, PPW)], pos_hbm.at[pl.ds(base, PPW)])
    pltpu.sync_copy(dis_v.at[pl.ds(PPW, PPW)], neg_hbm.at[pl.ds(base, PPW)])

    lane = jnp.arange(16, dtype=jnp.int32)
    part_v[...] = jnp.where(lane == 0, jnp.sum((psum + nsum) * LOSS_SCALE), 0.0)
    pltpu.sync_copy(part_v.at[pl.ds(0, 8)], shared.at[pl.ds(sid * 8, 8)])
    plsc.subcore_barrier()

    @pl.when(sid == 0)
    def _():
        pltpu.sync_copy(shared, red_v)
        mask = lax.rem(lane, 8) == 0
        tot = jnp.zeros((16,), jnp.float32)
        for k in range(8):
            v = red_v[pl.ds(k * 16, 16)]
            tot = tot + jnp.where(mask, v, 0.0)
        part_v[...] = jnp.where(lane == 0, jnp.sum(tot), 0.0)
        pltpu.sync_copy(part_v.at[pl.ds(0, 8)], parts_hbm.at[pl.ds(cid * 8, 8)])


@jax.jit
def _sc_call(fixt, movt, pts):
    mesh = plsc.VectorSubcoreMesh(core_axis_name="c", subcore_axis_name="s")
    relayout = functools.partial(
        pl.kernel,
        mesh=mesh,
        compiler_params=pltpu.CompilerParams(
            use_tc_tiling_on_sc=True, needs_layout_passes=False
        ),
        out_type=[
            jax.ShapeDtypeStruct((C * VOL,), jnp.float32),
            jax.ShapeDtypeStruct((C * VOL,), jnp.float32),
        ],
        scratch_types=[
            pltpu.VMEM((CHROWS, SX), jnp.float32),
            pltpu.VMEM((CHROWS, SX), jnp.float32),
            pltpu.VMEM((HWORDS,), jnp.float32),
            pltpu.VMEM((HWORDS,), jnp.float32),
            pltpu.SemaphoreType.DMA,
            pltpu.SemaphoreType.DMA,
            pltpu.SemaphoreType.DMA,
        ],
    )(_relayout_body)
    fix_flat, mov_flat = relayout(fixt, movt)

    gather = functools.partial(
        pl.kernel,
        mesh=mesh,
        compiler_params=pltpu.CompilerParams(needs_layout_passes=False),
        out_type=[
            jax.ShapeDtypeStruct((NC * 8,), jnp.float32),
            jax.ShapeDtypeStruct((NPTS,), jnp.float32),
            jax.ShapeDtypeStruct((NPTS,), jnp.float32),
        ],
        scratch_types=[
            pltpu.VMEM((9, PPW), jnp.int32),
            pltpu.VMEM((3, NROW, 128), jnp.int32),
            pltpu.VMEM((3, NROW, 128), jnp.float32),
            pltpu.VMEM((2 * PPW,), jnp.float32),
            pltpu.VMEM((16,), jnp.float32),
            pltpu.VMEM((128,), jnp.float32),
            pltpu.VMEM_SHARED((128,), jnp.float32),
            pltpu.SemaphoreType.DMA,
        ],
    )(_gather_body)
    return gather(fix_flat, mov_flat, pts)


def kernel(fix_image_feature, moving_image_feature, fixed_points,
           positive_points, negative_points, x_shard, y_shard, z_shard):
    fixt = jnp.transpose(fix_image_feature, (0, 1, 3, 4, 2)).reshape(NROWS, SX)
    movt = jnp.transpose(moving_image_feature, (0, 1, 3, 4, 2)).reshape(NROWS, SX)
    pts = jnp.concatenate(
        [fixed_points.T, positive_points.T, negative_points.T], axis=0
    ).astype(jnp.int32).reshape(-1)
    parts, pos_dis, neg_dis = _sc_call(fixt, movt, pts)
    loss = parts[0] + parts[8]
    return loss, pos_dis, neg_dis


# speedup vs baseline: 1.2255x; 1.0233x over previous
import functools

import jax
import jax.numpy as jnp
from jax import lax
from jax.experimental import pallas as pl
from jax.experimental.pallas import tpu as pltpu
from jax.experimental.pallas import tpu_sc as plsc

C = 64
NPTS = 2048
SX, SY, SZ = 100, 88, 80
VOL = SX * SY * SZ
NROWS = C * SY * SZ
PADVOL = SY * SZ * 128
NC, NS = 2, 16
NW = NC * NS
PPW = NPTS // NW
NV = PPW // 16
NROW = 32
LOSS_SCALE = 100.0 / (4.0 * NPTS)

ROWS_PER_TILE = NROWS // NW
CHROWS = 320
NCHUNK = ROWS_PER_TILE // CHROWS
HROWS = CHROWS // 2
HWORDS = HROWS * SX
ROW_OFFS = (0, 16, 32, 48, 64, 80, 84)


def _relayout_body(fixt_hbm, movt_hbm, fixo_hbm, movo_hbm,
                   in0, in1, out0, out1, sem_in0, sem_in1,
                   sem_out0, sem_out1):
    cid = lax.axis_index("c")
    sid = lax.axis_index("s")
    wid = cid * NS + sid
    row0 = wid * ROWS_PER_TILE

    def repack(src_buf, half, dst_buf):
        UR = 4
        r_base = half * HROWS

        @plsc.parallel_loop(0, HROWS, step=UR, unroll=2)
        def _rows(r0):
            vals = [
                src_buf[r_base + r0 + u, pl.ds(off, 16)]
                for u in range(UR)
                for off in ROW_OFFS
            ]
            k = 0
            for u in range(UR):
                o = (r0 + u) * SX
                for off in ROW_OFFS:
                    dst_buf[pl.ds(o + off, 16)] = vals[k]
                    k += 1

    for src, dst in ((fixt_hbm, fixo_hbm), (movt_hbm, movo_hbm)):
        def start_in(c, b, sem):
            pltpu.async_copy(src.at[pl.ds(row0 + c * CHROWS, CHROWS)], b, sem)

        def wait_in(b, sem):
            pltpu.make_async_copy(src.at[pl.ds(row0, CHROWS)], b, sem).wait()

        def start_out(c, half, b, sem):
            off = (row0 + c * CHROWS + half * HROWS) * SX
            pltpu.async_copy(b, dst.at[pl.ds(off, HWORDS)], sem)

        def wait_out(b, sem):
            pltpu.make_async_copy(b, dst.at[pl.ds(row0, HWORDS)], sem).wait()

        def do_half(c, half, src_buf, out_buf, sem, first):
            if first:
                @pl.when(c > 0)
                def _():
                    wait_out(out_buf, sem)
            else:
                wait_out(out_buf, sem)
            repack(src_buf, half, out_buf)
            start_out(c, half, out_buf, sem)

        start_in(0, in0, sem_in0)
        start_in(1, in1, sem_in1)

        def pair(m, _):
            c0 = 2 * m
            wait_in(in0, sem_in0)
            do_half(c0, 0, in0, out0, sem_out0, True)
            do_half(c0, 1, in0, out1, sem_out1, True)

            @pl.when(c0 + 2 < NCHUNK)
            def _():
                start_in(c0 + 2, in0, sem_in0)

            wait_in(in1, sem_in1)
            do_half(c0 + 1, 0, in1, out0, sem_out0, False)
            do_half(c0 + 1, 1, in1, out1, sem_out1, False)

            @pl.when(c0 + 3 < NCHUNK)
            def _():
                start_in(c0 + 3, in1, sem_in1)

            return 0

        lax.fori_loop(0, NCHUNK // 2, pair, 0)
        wait_out(out0, sem_out0)
        wait_out(out1, sem_out1)


def _sqrt16(x):
    i = plsc.bitcast(x, jnp.int32)
    y = plsc.bitcast(jnp.int32(0x5F3759DF) - (i >> 1), jnp.float32)
    for _ in range(3):
        y = y * (1.5 - 0.5 * x * y * y)
    return x * y


def _gather_body(fix_hbm, mov_hbm, pts_hbm,
                 parts_hbm, pos_hbm, neg_hbm,
                 pts_v, idx_v, rows_v, dis_v, part_v, red_v, shared, sem):
    cid = lax.axis_index("c")
    sid = lax.axis_index("s")
    wid = cid * NS + sid
    base = wid * PPW

    for r in range(9):
        pltpu.sync_copy(pts_hbm.at[pl.ds(r * NPTS + base, PPW)], pts_v.at[r])

    svecs = []
    for g in range(3):
        for i in range(NV):
            sl = pl.ds(i * 16, 16)
            x = lax.rem(pts_v[3 * g + 0, sl], SX)
            y = lax.rem(pts_v[3 * g + 1, sl], SY)
            z = lax.rem(pts_v[3 * g + 2, sl], SZ)
            svecs.append((y * SZ + z) * SX + x)

    def build_row(j, sv):
        c0 = (2 * j) * VOL
        for g in range(3):
            row = idx_v.at[g, j]
            for half in range(2):
                coff = c0 + half * VOL
                for i in range(NV):
                    row[pl.ds(half * PPW + i * 16, 16)] = sv[NV * g + i] + coff
        return sv

    lax.fori_loop(0, NROW, build_row, tuple(svecs))

    def fire(j, _):
        pltpu.async_copy(fix_hbm.at[idx_v.at[0, j]], rows_v.at[0, j], sem)
        pltpu.async_copy(mov_hbm.at[idx_v.at[1, j]], rows_v.at[1, j], sem)
        pltpu.async_copy(mov_hbm.at[idx_v.at[2, j]], rows_v.at[2, j], sem)
        return 0

    lax.fori_loop(0, NROW, fire, 0)

    def drain(j, _):
        pltpu.make_async_copy(fix_hbm.at[idx_v.at[0, j]], rows_v.at[0, j], sem).wait()
        pltpu.make_async_copy(mov_hbm.at[idx_v.at[1, j]], rows_v.at[1, j], sem).wait()
        pltpu.make_async_copy(mov_hbm.at[idx_v.at[2, j]], rows_v.at[2, j], sem).wait()
        return 0

    lax.fori_loop(0, NROW, drain, 0)

    def accum(j, accs):
        accs = list(accs)
        for half in range(2):
            for i in range(NV):
                sl = pl.ds(half * PPW + i * 16, 16)
                a = rows_v[0, j, sl]
                p = rows_v[1, j, sl]
                n = rows_v[2, j, sl]
                dp = a - p
                dn = a - n
                accs[i] = accs[i] + dp * dp
                accs[NV + i] = accs[NV + i] + dn * dn
        return tuple(accs)

    zeros = jnp.zeros((16,), jnp.float32)
    accs = lax.fori_loop(0, NROW, accum, (zeros,) * (2 * NV))

    psum = zeros
    nsum = zeros
    for i in range(NV):
        pos_d2 = accs[i]
        neg_d2 = accs[NV + i]
        dis_v[pl.ds(i * 16, 16)] = _sqrt16(pos_d2)
        neg_dis = _sqrt16(neg_d2)
        dis_v[pl.ds(PPW + i * 16, 16)] = neg_dis
        hinge = jnp.maximum(1.0 - neg_dis, 0.0)
        psum = psum + pos_d2
        nsum = nsum + hinge * hinge

    pltpu.sync_copy(dis_v.at[pl.ds(0, PPW)], pos_hbm.at[pl.ds(base, PPW)])
    pltpu.sync_copy(dis_v.at[pl.ds(PPW, PPW)], neg_hbm.at[pl.ds(base, PPW)])

    lane = jnp.arange(16, dtype=jnp.int32)
    part_v[...] = jnp.where(lane == 0, jnp.sum((psum + nsum) * LOSS_SCALE), 0.0)
    pltpu.sync_copy(part_v.at[pl.ds(0, 8)], shared.at[pl.ds(sid * 8, 8)])
    plsc.subcore_barrier()

    @pl.when(sid == 0)
    def _():
        pltpu.sync_copy(shared, red_v)
        mask = lax.rem(lane, 8) == 0
        tot = jnp.zeros((16,), jnp.float32)
        for k in range(8):
            v = red_v[pl.ds(k * 16, 16)]
            tot = tot + jnp.where(mask, v, 0.0)
        part_v[...] = jnp.where(lane == 0, jnp.sum(tot), 0.0)
        pltpu.sync_copy(part_v.at[pl.ds(0, 8)], parts_hbm.at[pl.ds(cid * 8, 8)])


@jax.jit
def _sc_call(fixt, movt, pts):
    mesh = plsc.VectorSubcoreMesh(core_axis_name="c", subcore_axis_name="s")
    relayout = functools.partial(
        pl.kernel,
        mesh=mesh,
        compiler_params=pltpu.CompilerParams(
            use_tc_tiling_on_sc=True, needs_layout_passes=False
        ),
        out_type=[
            jax.ShapeDtypeStruct((C * VOL,), jnp.float32),
            jax.ShapeDtypeStruct((C * VOL,), jnp.float32),
        ],
        scratch_types=[
            pltpu.VMEM((CHROWS, SX), jnp.float32),
            pltpu.VMEM((CHROWS, SX), jnp.float32),
            pltpu.VMEM((HWORDS,), jnp.float32),
            pltpu.VMEM((HWORDS,), jnp.float32),
            pltpu.SemaphoreType.DMA,
            pltpu.SemaphoreType.DMA,
            pltpu.SemaphoreType.DMA,
            pltpu.SemaphoreType.DMA,
        ],
    )(_relayout_body)
    fix_flat, mov_flat = relayout(fixt, movt)

    gather = functools.partial(
        pl.kernel,
        mesh=mesh,
        compiler_params=pltpu.CompilerParams(needs_layout_passes=False),
        out_type=[
            jax.ShapeDtypeStruct((NC * 8,), jnp.float32),
            jax.ShapeDtypeStruct((NPTS,), jnp.float32),
            jax.ShapeDtypeStruct((NPTS,), jnp.float32),
        ],
        scratch_types=[
            pltpu.VMEM((9, PPW), jnp.int32),
            pltpu.VMEM((3, NROW, 128), jnp.int32),
            pltpu.VMEM((3, NROW, 128), jnp.float32),
            pltpu.VMEM((2 * PPW,), jnp.float32),
            pltpu.VMEM((16,), jnp.float32),
            pltpu.VMEM((128,), jnp.float32),
            pltpu.VMEM_SHARED((128,), jnp.float32),
            pltpu.SemaphoreType.DMA,
        ],
    )(_gather_body)
    return gather(fix_flat, mov_flat, pts)


def kernel(fix_image_feature, moving_image_feature, fixed_points,
           positive_points, negative_points, x_shard, y_shard, z_shard):
    fixt = jnp.transpose(fix_image_feature, (0, 1, 3, 4, 2)).reshape(NROWS, SX)
    movt = jnp.transpose(moving_image_feature, (0, 1, 3, 4, 2)).reshape(NROWS, SX)
    pts = jnp.concatenate(
        [fixed_points.T, positive_points.T, negative_points.T], axis=0
    ).astype(jnp.int32).reshape(-1)
    parts, pos_dis, neg_dis = _sc_call(fixt, movt, pts)
    loss = parts[0] + parts[8]
    return loss, pos_dis, neg_dis
